# baseline (device time: 9252 ns/iter reference)
import jax
import jax.numpy as jnp
from jax import lax
from jax.experimental import pallas as pl
from jax.experimental.pallas import tpu as pltpu

N_GLOBAL = 1024.0
EPS = 1e-5
N_CHUNKS = 2


def kernel(x, gamma, beta):
    m, n_loc = x.shape
    mc = m // N_CHUNKS

    def body(
        x_hbm,
        g_hbm,
        b_hbm,
        out_hbm,
        xv_ref,
        g_v,
        b_v,
        out_v,
        my_stats,
        peer_stats,
        load_sems,
        gb_sems,
        store_sems,
        send_sem,
        recv_sem,
    ):
        my_x = lax.axis_index("x")
        my_y = lax.axis_index("y")
        peer = (my_x, 1 - my_y)

        barrier_sem = pltpu.get_barrier_semaphore()
        pl.semaphore_signal(
            barrier_sem, inc=1, device_id=peer, device_id_type=pl.DeviceIdType.MESH
        )

        def load_chunk(c):
            return pltpu.make_async_copy(
                x_hbm.at[pl.ds(c * mc, mc), :],
                xv_ref.at[pl.ds(c * mc, mc), :],
                load_sems.at[c],
            )

        loads = [load_chunk(c) for c in range(N_CHUNKS)]
        for cp in loads:
            cp.start()
        cp_g = pltpu.make_async_copy(g_hbm, g_v, gb_sems.at[0])
        cp_b = pltpu.make_async_copy(b_hbm, b_v, gb_sems.at[1])
        cp_g.start()
        cp_b.start()

        for c in range(N_CHUNKS):
            loads[c].wait()
            xv = xv_ref[pl.ds(c * mc, mc), :]
            s = jnp.sum(xv, axis=1, keepdims=True)
            sq = jnp.sum(xv * xv, axis=1, keepdims=True)
            my_stats[:, pl.ds(c * mc, mc)] = jnp.concatenate([s, sq], axis=1).T

        pl.semaphore_wait(barrier_sem, 1)

        rdma = pltpu.make_async_remote_copy(
            src_ref=my_stats,
            dst_ref=peer_stats,
            send_sem=send_sem,
            recv_sem=recv_sem,
            device_id=peer,
            device_id_type=pl.DeviceIdType.MESH,
        )
        rdma.start()

        cp_g.wait()
        cp_b.wait()
        g = g_v[:].reshape(1, n_loc)
        b = b_v[:].reshape(1, n_loc)

        rdma.wait_recv()

        def store_chunk(c):
            return pltpu.make_async_copy(
                out_v.at[pl.ds(c * mc, mc), :],
                out_hbm.at[pl.ds(c * mc, mc), :],
                store_sems.at[c],
            )

        for c in range(N_CHUNKS):
            tot2 = (
                my_stats[:, pl.ds(c * mc, mc)] + peer_stats[:, pl.ds(c * mc, mc)]
            ).T
            mean = tot2[:, 0:1] / N_GLOBAL
            var = tot2[:, 1:2] / N_GLOBAL - mean * mean
            inv = lax.rsqrt(var + EPS)
            xv = xv_ref[pl.ds(c * mc, mc), :]
            out = (xv - mean) * inv * g + b
            out_v[pl.ds(c * mc, mc), :] = out.astype(out_v.dtype)
            store_chunk(c).start()

        for c in range(N_CHUNKS):
            store_chunk(c).wait()
        rdma.wait_send()

    return pl.pallas_call(
        body,
        out_shape=jax.ShapeDtypeStruct((m, n_loc), jnp.bfloat16),
        in_specs=[pl.BlockSpec(memory_space=pl.ANY)] * 3,
        out_specs=pl.BlockSpec(memory_space=pl.ANY),
        scratch_shapes=[
            pltpu.VMEM((m, n_loc), jnp.float32),
            pltpu.VMEM((n_loc,), jnp.float32),
            pltpu.VMEM((n_loc,), jnp.float32),
            pltpu.VMEM((m, n_loc), jnp.bfloat16),
            pltpu.VMEM((2, m), jnp.float32),
            pltpu.VMEM((2, m), jnp.float32),
            pltpu.SemaphoreType.DMA((N_CHUNKS,)),
            pltpu.SemaphoreType.DMA((2,)),
            pltpu.SemaphoreType.DMA((N_CHUNKS,)),
            pltpu.SemaphoreType.DMA,
            pltpu.SemaphoreType.DMA,
        ],
        compiler_params=pltpu.CompilerParams(collective_id=0),
    )(x, gamma, beta)


# device time: 8474 ns/iter; 1.0918x vs baseline; 1.0918x over previous
import jax
import jax.numpy as jnp
from jax import lax
from jax.experimental import pallas as pl
from jax.experimental.pallas import tpu as pltpu

N_GLOBAL = 1024.0
EPS = 1e-5


def kernel(x, gamma, beta):
    m, n_loc = x.shape

    def body(x_ref, g_ref, b_ref, out_ref, my_stats, peer_stats, send_sem, recv_sem):
        my_x = lax.axis_index("x")
        my_y = lax.axis_index("y")
        peer = (my_x, 1 - my_y)

        barrier_sem = pltpu.get_barrier_semaphore()
        pl.semaphore_signal(
            barrier_sem, inc=1, device_id=peer, device_id_type=pl.DeviceIdType.MESH
        )

        xv = x_ref[:, :]
        s = jnp.sum(xv, axis=1, keepdims=True)
        sq = jnp.sum(xv * xv, axis=1, keepdims=True)
        my_stats[:, :] = jnp.concatenate([s, sq], axis=1).T

        pl.semaphore_wait(barrier_sem, 1)

        rdma = pltpu.make_async_remote_copy(
            src_ref=my_stats,
            dst_ref=peer_stats,
            send_sem=send_sem,
            recv_sem=recv_sem,
            device_id=peer,
            device_id_type=pl.DeviceIdType.MESH,
        )
        rdma.start()
        rdma.wait_recv()

        tot2 = (my_stats[:, :] + peer_stats[:, :]).T
        mean = tot2[:, 0:1] / N_GLOBAL
        var = tot2[:, 1:2] / N_GLOBAL - mean * mean
        inv = lax.rsqrt(var + EPS)
        g = g_ref[:, :]
        b = b_ref[:, :]
        out = (xv - mean) * inv * g + b
        out_ref[:, :] = out.astype(out_ref.dtype)

        rdma.wait_send()

    return pl.pallas_call(
        body,
        out_shape=jax.ShapeDtypeStruct((m, n_loc), jnp.bfloat16),
        in_specs=[pl.BlockSpec(memory_space=pltpu.VMEM)] * 3,
        out_specs=pl.BlockSpec(memory_space=pltpu.VMEM),
        scratch_shapes=[
            pltpu.VMEM((2, m), jnp.float32),
            pltpu.VMEM((2, m), jnp.float32),
            pltpu.SemaphoreType.DMA,
            pltpu.SemaphoreType.DMA,
        ],
        compiler_params=pltpu.CompilerParams(collective_id=0),
    )(x, gamma.reshape(1, n_loc), beta.reshape(1, n_loc))
